# baseline (speedup 1.0000x reference)
import functools

import jax
import jax.numpy as jnp
from jax import lax
from jax.experimental import pallas as pl
from jax.experimental.pallas import tpu as pltpu
from jax.experimental.pallas import tpu_sc as plsc

VOCAB = 1000000
D_MODEL = 64
NJ = 50
NI = 16384
NC = 2
NS = 16
NW = NC * NS
C = 256
NBLK = NI // C
OUT_FLAT = NJ * 8 * 128 * 8 * 128

_mesh = plsc.VectorSubcoreMesh(core_axis_name="c", subcore_axis_name="s")

_scratch = (
    [pltpu.VMEM((NJ, C), jnp.int32) for _ in range(2)]
    + [pltpu.VMEM((C, D_MODEL), jnp.float32) for _ in range(2)]
    + [pltpu.VMEM((8 * 2 * 8 * 128,), jnp.float32) for _ in range(2)]
    + [pltpu.SemaphoreType.DMA for _ in range(2)]
    + [pltpu.SemaphoreType.DMA for _ in range(2)]
)


@functools.partial(
    pl.kernel,
    mesh=_mesh,
    out_type=jax.ShapeDtypeStruct((OUT_FLAT,), jnp.float32),
    scratch_types=_scratch,
    compiler_params=pltpu.CompilerParams(
        use_tc_tiling_on_sc=False, needs_layout_passes=False
    ),
)
def _gather_t(xt_hbm, table_hbm, out_hbm, idx0, idx1, rows0, rows1,
              pk0, pk1, g0, g1, o0, o1):
    wid = lax.axis_index("s") * NC + lax.axis_index("c")
    idx_v = [idx0, idx1]
    rows = [rows0, rows1]
    pk = [pk0, pk1]
    gsem = [g0, g1]
    osem = [o0, o1]
    blk = [wid, wid + NW]

    for s in range(2):
        pltpu.sync_copy(xt_hbm.at[:, pl.ds(blk[s] * C, C)], idx_v[s])

    lane = lax.iota(jnp.int32, 16)
    voff = [(2 * k + lane // 8) * 2048 + (lane % 8) * 128 for k in range(4)]

    def fire_gather(s, j):
        pltpu.make_async_copy(
            table_hbm.at[idx_v[s].at[j]], rows[s], gsem[s]
        ).start()

    def wait_gather(s, j):
        pltpu.make_async_copy(
            table_hbm.at[idx_v[s].at[j]], rows[s], gsem[s]
        ).wait()

    def store_off(s, j, tr):
        return ((j * 8 + tr) * 128 + 2 * blk[s]) * 1024

    def fire_store(s, j):
        for tr in range(8):
            pltpu.make_async_copy(
                pk[s].at[pl.ds(tr * 2048, 2048)],
                out_hbm.at[pl.ds(store_off(s, j, tr), 2048)],
                osem[s],
            ).start()

    def wait_store(s, j):
        for tr in range(8):
            pltpu.make_async_copy(
                pk[s].at[pl.ds(tr * 2048, 2048)],
                out_hbm.at[pl.ds(store_off(s, j, tr), 2048)],
                osem[s],
            ).wait()

    def transpose(s):
        @plsc.parallel_loop(0, 16, unroll=2)
        def _trans(g):
            tcb = g // 8
            is0 = (g - tcb * 8) * 16
            base = tcb * 1024 + is0
            dst = pk[s].at[pl.ds(base, 7056)]
            for p in range(16):
                n = tcb * 128 + is0 + p
                for k in range(2):
                    vals = rows[s][n, pl.ds(16 * k, 16)]
                    plsc.store_scatter(dst, [voff[k] + p], vals)
            row_idx = lane + (tcb * 128 + is0)
            for d in range(32, 64):
                tr, ds_ = d // 8, d % 8
                vals = plsc.load_gather(
                    rows[s], [row_idx, jnp.full((16,), d, jnp.int32)]
                )
                off = tr * 2048 + tcb * 1024 + ds_ * 128 + is0
                pk[s][pl.ds(off, 16)] = vals

    fire_gather(0, 0)
    fire_gather(1, 0)

    def round_body(r, first, last):
        for s in range(2):
            wait_gather(s, r)
            if not first:
                wait_store(s, r - 1)
            transpose(s)
            fire_store(s, r)
            if not last:
                fire_gather(s, r + 1)

    def body(r, carry):
        round_body(r, False, False)
        return carry

    round_body(0, True, False)
    lax.fori_loop(1, NJ - 1, body, 0)
    round_body(NJ - 1, False, True)
    for s in range(2):
        wait_store(s, NJ - 1)


def kernel(x, table):
    xt = x.T.astype(jnp.int32)
    out_flat = _gather_t(xt, table)
    out5 = out_flat.reshape(NJ, 8, 128, 8, 128)
    return lax.reshape(out5, (NI, NJ, D_MODEL), dimensions=(2, 4, 0, 1, 3))


# speedup vs baseline: 1.0114x; 1.0114x over previous
import functools

import jax
import jax.numpy as jnp
from jax import lax
from jax.experimental import pallas as pl
from jax.experimental.pallas import tpu as pltpu
from jax.experimental.pallas import tpu_sc as plsc

VOCAB = 1000000
D_MODEL = 64
NJ = 50
NI = 16384
NC = 2
NS = 16
NW = NC * NS
C = 256
NBLK = NI // C
OUT_FLAT = NJ * 8 * 128 * 8 * 128

_mesh = plsc.VectorSubcoreMesh(core_axis_name="c", subcore_axis_name="s")

_scratch = (
    [pltpu.VMEM((NJ, C), jnp.int32) for _ in range(2)]
    + [pltpu.VMEM((C, D_MODEL), jnp.float32) for _ in range(2)]
    + [pltpu.VMEM((8 * 2 * 8 * 128,), jnp.float32) for _ in range(2)]
    + [pltpu.SemaphoreType.DMA for _ in range(2)]
    + [pltpu.SemaphoreType.DMA for _ in range(2)]
)


@functools.partial(
    pl.kernel,
    mesh=_mesh,
    out_type=jax.ShapeDtypeStruct((OUT_FLAT,), jnp.float32),
    scratch_types=_scratch,
    compiler_params=pltpu.CompilerParams(
        use_tc_tiling_on_sc=False, needs_layout_passes=False
    ),
)
def _gather_t(xt_hbm, table_hbm, out_hbm, idx0, idx1, rows0, rows1,
              pk0, pk1, g0, g1, o0, o1):
    wid = lax.axis_index("s") * NC + lax.axis_index("c")
    idx_v = [idx0, idx1]
    rows = [rows0, rows1]
    pk = [pk0, pk1]
    gsem = [g0, g1]
    osem = [o0, o1]
    blk = [wid, wid + NW]

    for s in range(2):
        pltpu.sync_copy(xt_hbm.at[:, pl.ds(blk[s] * C, C)], idx_v[s])

    lane = lax.iota(jnp.int32, 16)
    voff = [(2 * k + lane // 8) * 2048 + (lane % 8) * 128 for k in range(4)]

    def fire_gather(s, j):
        pltpu.make_async_copy(
            table_hbm.at[idx_v[s].at[j]], rows[s], gsem[s]
        ).start()

    def wait_gather(s, j):
        pltpu.make_async_copy(
            table_hbm.at[idx_v[s].at[j]], rows[s], gsem[s]
        ).wait()

    def store_off(s, j, tr):
        return ((j * 8 + tr) * 128 + 2 * blk[s]) * 1024

    def fire_store(s, j):
        for tr in range(8):
            pltpu.make_async_copy(
                pk[s].at[pl.ds(tr * 2048, 2048)],
                out_hbm.at[pl.ds(store_off(s, j, tr), 2048)],
                osem[s],
            ).start()

    def wait_store(s, j):
        for tr in range(8):
            pltpu.make_async_copy(
                pk[s].at[pl.ds(tr * 2048, 2048)],
                out_hbm.at[pl.ds(store_off(s, j, tr), 2048)],
                osem[s],
            ).wait()

    def transpose(s):
        @plsc.parallel_loop(0, 16, unroll=4)
        def _trans(g):
            tcb = g // 8
            is0 = (g - tcb * 8) * 16
            base = tcb * 1024 + is0
            dst = pk[s].at[pl.ds(base, 7056)]
            for p in range(16):
                n = tcb * 128 + is0 + p
                for k in range(2):
                    vals = rows[s][n, pl.ds(16 * k, 16)]
                    plsc.store_scatter(dst, [voff[k] + p], vals)
            row_idx = lane + (tcb * 128 + is0)
            for d in range(32, 64):
                tr, ds_ = d // 8, d % 8
                vals = plsc.load_gather(
                    rows[s], [row_idx, jnp.full((16,), d, jnp.int32)]
                )
                off = tr * 2048 + tcb * 1024 + ds_ * 128 + is0
                pk[s][pl.ds(off, 16)] = vals

    fire_gather(0, 0)
    fire_gather(1, 0)

    def round_body(r, first, last):
        for s in range(2):
            wait_gather(s, r)
            if not first:
                wait_store(s, r - 1)
            transpose(s)
            fire_store(s, r)
            if not last:
                fire_gather(s, r + 1)

    def body(r, carry):
        round_body(r, False, False)
        return carry

    round_body(0, True, False)
    lax.fori_loop(1, NJ - 1, body, 0)
    round_body(NJ - 1, False, True)
    for s in range(2):
        wait_store(s, NJ - 1)


def kernel(x, table):
    xt = x.T.astype(jnp.int32)
    out_flat = _gather_t(xt, table)
    out5 = out_flat.reshape(NJ, 8, 128, 8, 128)
    return lax.reshape(out5, (NI, NJ, D_MODEL), dimensions=(2, 4, 0, 1, 3))
